# Initial kernel scaffold; baseline (speedup 1.0000x reference)
import numpy as np
import jax
import jax.numpy as jnp
from jax import lax
from jax.experimental import pallas as pl
from jax.experimental.pallas import tpu as pltpu
from jax.experimental.pallas import tpu_sc as plsc

N = 10000
E = 320000
H = 128
NC = 2
NS = 16
NW = NC * NS
CH = 128
NCHUNK = 79
EPW = NCHUNK * CH
EP = NW * EPW
NPT = 10016
TPS = NPT // NS
MM_BLK = 2048
NLAYERS = 6


def _sc_gather_body(p_hbm, q_hbm, dsti, srci, pd_out, qs_out,
                    idxd, idxs, bufa, bufb, sema, semb):
    c = lax.axis_index("c")
    s = lax.axis_index("s")
    wid = s * NC + c
    base = wid * EPW
    pltpu.sync_copy(dsti.at[wid], idxd)
    pltpu.sync_copy(srci.at[wid], idxs)

    def chunk(j, carry):
        cpa = pltpu.async_copy(p_hbm.at[idxd.at[j]], bufa, sema)
        cpb = pltpu.async_copy(q_hbm.at[idxs.at[j]], bufb, semb)
        cpa.wait()
        cpb.wait()
        pltpu.sync_copy(bufa, pd_out.at[pl.ds(base + j * CH, CH), :])
        pltpu.sync_copy(bufb, qs_out.at[pl.ds(base + j * CH, CH), :])
        return carry

    lax.fori_loop(0, NCHUNK, chunk, 0)


_gather = pl.kernel(
    _sc_gather_body,
    out_type=[jax.ShapeDtypeStruct((EP, H), jnp.float32),
              jax.ShapeDtypeStruct((EP, H), jnp.float32)],
    mesh=plsc.VectorSubcoreMesh(core_axis_name="c", subcore_axis_name="s"),
    scratch_types=[pltpu.VMEM((NCHUNK, CH), jnp.int32),
                   pltpu.VMEM((NCHUNK, CH), jnp.int32),
                   pltpu.VMEM((CH, H), jnp.float32),
                   pltpu.VMEM((CH, H), jnp.float32),
                   pltpu.SemaphoreType.DMA,
                   pltpu.SemaphoreType.DMA],
)


def _sc_scatter_body(m2_hbm, dsti, zeros_hbm, part_out, idxd, mbuf, acc):
    c = lax.axis_index("c")
    s = lax.axis_index("s")
    wid = s * NC + c
    pltpu.sync_copy(zeros_hbm, acc.at[pl.ds(s * TPS, TPS), :])
    plsc.subcore_barrier()
    pltpu.sync_copy(dsti.at[wid], idxd)

    def chunk(j, carry):
        pltpu.sync_copy(m2_hbm.at[pl.ds(wid * EPW + j * CH, CH), :], mbuf)
        pltpu.sync_copy(mbuf, acc.at[idxd.at[j]], add=True)
        return carry

    lax.fori_loop(0, NCHUNK, chunk, 0)
    plsc.subcore_barrier()
    pltpu.sync_copy(acc.at[pl.ds(s * TPS, TPS), :],
                    part_out.at[c, pl.ds(s * TPS, TPS), :])


_scatter = pl.kernel(
    _sc_scatter_body,
    out_type=jax.ShapeDtypeStruct((NC, NPT, H), jnp.float32),
    mesh=plsc.VectorSubcoreMesh(core_axis_name="c", subcore_axis_name="s"),
    scratch_types=[pltpu.VMEM((NCHUNK, CH), jnp.int32),
                   pltpu.VMEM((CH, H), jnp.float32),
                   pltpu.VMEM_SHARED((NPT, H), jnp.float32)],
)


def _sc_count_body(dsti, ones_hbm, zeros_hbm, cnt_out, idxd, onesb, acc):
    c = lax.axis_index("c")
    s = lax.axis_index("s")
    wid = s * NC + c
    pltpu.sync_copy(zeros_hbm, acc.at[pl.ds(s * TPS, TPS), :])
    pltpu.sync_copy(ones_hbm, onesb)
    plsc.subcore_barrier()
    pltpu.sync_copy(dsti.at[wid], idxd)

    def chunk(j, carry):
        pltpu.sync_copy(onesb, acc.at[idxd.at[j]], add=True)
        return carry

    lax.fori_loop(0, NCHUNK, chunk, 0)
    plsc.subcore_barrier()
    pltpu.sync_copy(acc.at[pl.ds(s * TPS, TPS), :],
                    cnt_out.at[c, pl.ds(s * TPS, TPS), :])


_count = pl.kernel(
    _sc_count_body,
    out_type=jax.ShapeDtypeStruct((NC, NPT, 16), jnp.float32),
    mesh=plsc.VectorSubcoreMesh(core_axis_name="c", subcore_axis_name="s"),
    scratch_types=[pltpu.VMEM((NCHUNK, CH), jnp.int32),
                   pltpu.VMEM((CH, 16), jnp.float32),
                   pltpu.VMEM_SHARED((NPT, 16), jnp.float32)],
)


def _mm_body(pd_ref, qs_ref, w_ref, b_ref, o_ref):
    g = jnp.maximum(pd_ref[...] + qs_ref[...], 0.0)
    m = jnp.dot(g, w_ref[...], preferred_element_type=jnp.float32)
    o_ref[...] = jnp.maximum(m + b_ref[0:1, :], 0.0)


_mm = pl.pallas_call(
    _mm_body,
    grid=(EP // MM_BLK,),
    in_specs=[pl.BlockSpec((MM_BLK, H), lambda i: (i, 0)),
              pl.BlockSpec((MM_BLK, H), lambda i: (i, 0)),
              pl.BlockSpec((H, H), lambda i: (0, 0)),
              pl.BlockSpec((8, H), lambda i: (0, 0))],
    out_specs=pl.BlockSpec((MM_BLK, H), lambda i: (i, 0)),
    out_shape=jax.ShapeDtypeStruct((EP, H), jnp.float32),
)


def _bn_in_kernel(x, g_ref, b_ref):
    mu = jnp.mean(x, axis=0)
    var = jnp.mean((x - mu) ** 2, axis=0)
    return g_ref[0:1, :] * (x - mu) / jnp.sqrt(var + 1e-5) + b_ref[0:1, :]


def _pq(h, vals, a_ref, b2_ref, w4_ref, bm1_ref, p_out, q_out):
    u, px, py, vv = vals
    svec = (u * w4_ref[0:1, :] + px * w4_ref[1:2, :] + py * w4_ref[2:3, :])
    p = jnp.dot(h, a_ref[...], preferred_element_type=jnp.float32)
    p = p + svec + vv * w4_ref[3:4, :] + bm1_ref[0:1, :]
    q = jnp.dot(h, b2_ref[...], preferred_element_type=jnp.float32) - svec
    p_out[0:N, :] = p
    p_out[N:NPT, :] = jnp.zeros((NPT - N, H), jnp.float32)
    q_out[0:N, :] = q
    q_out[N:NPT, :] = jnp.zeros((NPT - N, H), jnp.float32)


def _pre_body(scal_ref, w1_ref, b1_ref, g1_ref, be1_ref, w2_ref, b2_ref,
              g2_ref, be2_ref, a_ref, bb_ref, w4_ref, bm1_ref,
              h_out, p_out, q_out):
    scal = scal_ref[...]
    u, px, py, vv = (scal[:, 0:1], scal[:, 1:2], scal[:, 2:3], scal[:, 3:4])
    h = (u * w1_ref[0:1, :] + px * w1_ref[1:2, :] + py * w1_ref[2:3, :]
         + vv * w1_ref[3:4, :] + b1_ref[0:1, :])
    h = _bn_in_kernel(h, g1_ref, be1_ref)
    h = jnp.maximum(h, 0.0)
    h = jnp.dot(h, w2_ref[...], preferred_element_type=jnp.float32) + b2_ref[0:1, :]
    h = _bn_in_kernel(h, g2_ref, be2_ref)
    h_out[...] = h
    _pq(h, (u, px, py, vv), a_ref, bb_ref, w4_ref, bm1_ref, p_out, q_out)


_pre = pl.pallas_call(
    _pre_body,
    out_shape=[jax.ShapeDtypeStruct((N, H), jnp.float32),
               jax.ShapeDtypeStruct((NPT, H), jnp.float32),
               jax.ShapeDtypeStruct((NPT, H), jnp.float32)],
)


def _update_h(h_in_ref, parts_ref, cnts_ref, scal_ref, wu1h_ref, wu1a_ref,
              wu1v_ref, bu1_ref, wu2_ref, bu2_ref, g_ref, be_ref):
    h = h_in_ref[...]
    agg = parts_ref[0, 0:N, :] + parts_ref[1, 0:N, :]
    cnt = cnts_ref[0, 0:N, 0:1] + cnts_ref[1, 0:N, 0:1]
    agg = agg / jnp.maximum(cnt, 1.0)
    vv = scal_ref[:, 3:4]
    upd = (jnp.dot(h, wu1h_ref[...], preferred_element_type=jnp.float32)
           + jnp.dot(agg, wu1a_ref[...], preferred_element_type=jnp.float32)
           + vv * wu1v_ref[0:1, :] + bu1_ref[0:1, :])
    upd = jnp.maximum(upd, 0.0)
    upd = jnp.dot(upd, wu2_ref[...], preferred_element_type=jnp.float32) + bu2_ref[0:1, :]
    upd = jnp.maximum(upd, 0.0)
    return _bn_in_kernel(h + upd, g_ref, be_ref)


def _node_body(h_in_ref, parts_ref, cnts_ref, scal_ref, wu1h_ref, wu1a_ref,
               wu1v_ref, bu1_ref, wu2_ref, bu2_ref, g_ref, be_ref,
               a_ref, bb_ref, w4_ref, bm1_ref, h_out, p_out, q_out):
    hn = _update_h(h_in_ref, parts_ref, cnts_ref, scal_ref, wu1h_ref,
                   wu1a_ref, wu1v_ref, bu1_ref, wu2_ref, bu2_ref, g_ref, be_ref)
    h_out[...] = hn
    scal = scal_ref[...]
    vals = (scal[:, 0:1], scal[:, 1:2], scal[:, 2:3], scal[:, 3:4])
    _pq(hn, vals, a_ref, bb_ref, w4_ref, bm1_ref, p_out, q_out)


_node = pl.pallas_call(
    _node_body,
    out_shape=[jax.ShapeDtypeStruct((N, H), jnp.float32),
               jax.ShapeDtypeStruct((NPT, H), jnp.float32),
               jax.ShapeDtypeStruct((NPT, H), jnp.float32)],
)


def _final_body(h_in_ref, parts_ref, cnts_ref, scal_ref, wu1h_ref, wu1a_ref,
                wu1v_ref, bu1_ref, wu2_ref, bu2_ref, g_ref, be_ref,
                w1c_ref, b1c_ref, w2c_ref, b2c_ref, w3c_ref, b3c_ref, out_ref):
    hn = _update_h(h_in_ref, parts_ref, cnts_ref, scal_ref, wu1h_ref,
                   wu1a_ref, wu1v_ref, bu1_ref, wu2_ref, bu2_ref, g_ref, be_ref)
    z1 = jnp.maximum(jnp.dot(hn, w1c_ref[...], preferred_element_type=jnp.float32)
                     + b1c_ref[0:1, :], 0.0)
    z2 = jnp.maximum(jnp.dot(z1, w2c_ref[...], preferred_element_type=jnp.float32)
                     + b2c_ref[0:1, :], 0.0)
    z3 = jnp.dot(z2, w3c_ref[...], preferred_element_type=jnp.float32)
    out_ref[...] = (z3 + b3c_ref[0:1, :]) * 0.01


_final = pl.pallas_call(
    _final_body,
    out_shape=jax.ShapeDtypeStruct((N, H), jnp.float32),
)


def _conv_maps():
    r1, c1, o1, k1 = [], [], [], []
    for o in range(4):
        for j in range(38):
            for k in range(16):
                r1.append(3 * j + k); c1.append(o * 38 + j); o1.append(o); k1.append(k)
    r2, c2, o2, i2, k2 = [], [], [], [], []
    for o in range(8):
        for j in range(9):
            for i in range(4):
                for k in range(12):
                    r2.append(i * 38 + 3 * j + k); c2.append(o * 9 + j)
                    o2.append(o); i2.append(i); k2.append(k)
    r3, i3, k3 = [], [], []
    for i in range(8):
        for k in range(8):
            r3.append(i * 9 + k); i3.append(i); k3.append(k)
    f = lambda v: np.asarray(v, np.int32)
    return ((f(r1), f(c1), f(o1), f(k1)),
            (f(r2), f(c2), f(o2), f(i2), f(k2)),
            (f(r3), f(i3), f(k3)))


_C1, _C2, _C3 = _conv_maps()


def _row8(v):
    return jnp.broadcast_to(v.reshape(1, -1), (8, v.shape[-1]))


def kernel(x, pos, edge_index, batch, params):
    f32 = jnp.float32
    scal = jnp.concatenate(
        [x[:, 0:1], pos[:, 1:2], pos[:, 2:3], pos[:, 0:1]], axis=1).astype(f32)

    pad = EP - E
    srcp = jnp.concatenate([edge_index[0], jnp.full((pad,), N, jnp.int32)])
    dstp = jnp.concatenate([edge_index[1], jnp.full((pad,), N, jnp.int32)])
    src3 = srcp.reshape(NW, NCHUNK, CH)
    dst3 = dstp.reshape(NW, NCHUNK, CH)

    zeros_h = jnp.zeros((TPS, H), f32)
    zeros_c = jnp.zeros((TPS, 16), f32)
    ones_c = jnp.ones((CH, 16), f32)

    emb = params['emb']
    layers = params['layers']
    conv = params['conv']

    def msg_split(lp):
        wm1 = lp['wm1']
        return (wm1[0:H], wm1[H:2 * H], wm1[2 * H:2 * H + 4], _row8(lp['bm1']))

    a0, b0, w40, bm10 = msg_split(layers[0])
    h, p, q = _pre(scal, emb['w1'], _row8(emb['b1']), _row8(emb['g1']),
                   _row8(emb['be1']), emb['w2'], _row8(emb['b2']),
                   _row8(emb['g2']), _row8(emb['be2']), a0, b0, w40, bm10)

    cnts = _count(dst3, ones_c, zeros_c)

    w1c = jnp.zeros((H, 256), f32).at[_C1[0], _C1[1]].set(
        conv['w1'][_C1[2], 0, _C1[3]])
    b1c = jnp.zeros((256,), f32).at[np.arange(152, dtype=np.int32)].set(
        jnp.repeat(conv['b1'], 38))
    w2c = jnp.zeros((256, H), f32).at[_C2[0], _C2[1]].set(
        conv['w2'][_C2[2], _C2[3], _C2[4]])
    b2c = jnp.zeros((H,), f32).at[np.arange(72, dtype=np.int32)].set(
        jnp.repeat(conv['b2'], 9))
    w3c = jnp.zeros((H, H), f32).at[_C3[0], 0].set(conv['w3'][0, _C3[1], _C3[2]])
    b3c = jnp.broadcast_to(conv['b3'].reshape(1, 1), (8, H))

    out_full = None
    for i in range(NLAYERS):
        lp = layers[i]
        pd, qs = _gather(p, q, dst3, src3)
        m2 = _mm(pd, qs, lp['wm2'], _row8(lp['bm2']))
        parts = _scatter(m2, dst3, zeros_h)
        wu1 = lp['wu1']
        common = (h, parts, cnts, scal, wu1[0:H], wu1[H:2 * H],
                  wu1[2 * H:2 * H + 1], _row8(lp['bu1']), lp['wu2'],
                  _row8(lp['bu2']), _row8(lp['g']), _row8(lp['be']))
        if i < NLAYERS - 1:
            an, bn_, w4n, bm1n = msg_split(layers[i + 1])
            h, p, q = _node(*common, an, bn_, w4n, bm1n)
        else:
            out_full = _final(*common, w1c, _row8(b1c), w2c, _row8(b2c),
                              w3c, b3c)
    return out_full[:, 0:1]


# speedup vs baseline: 5.2385x; 5.2385x over previous
import numpy as np
import jax
import jax.numpy as jnp
from jax import lax
from jax.experimental import pallas as pl
from jax.experimental.pallas import tpu as pltpu
from jax.experimental.pallas import tpu_sc as plsc

N = 10000
E = 320000
H = 128
NC = 2
NS = 16
NW = NC * NS
CH = 128
NCHUNK = 79
EPW = NCHUNK * CH
EP = NW * EPW
NPT = 10112
TPS = NPT // NS
MM_BLK = 2048
NLAYERS = 6


def _sc_gather_body(p_hbm, q_hbm, dsti, srci, pd_out, qs_out,
                    idxd, idxs, bufa, bufb, sema, semb):
    c = lax.axis_index("c")
    s = lax.axis_index("s")
    wid = s * NC + c
    base = wid * EPW
    pltpu.sync_copy(dsti.at[wid], idxd)
    pltpu.sync_copy(srci.at[wid], idxs)

    def chunk(j, carry):
        cpa = pltpu.async_copy(p_hbm.at[idxd.at[j]], bufa, sema)
        cpb = pltpu.async_copy(q_hbm.at[idxs.at[j]], bufb, semb)
        cpa.wait()
        cpb.wait()
        pltpu.sync_copy(bufa, pd_out.at[pl.ds(base + j * CH, CH), :])
        pltpu.sync_copy(bufb, qs_out.at[pl.ds(base + j * CH, CH), :])
        return carry

    lax.fori_loop(0, NCHUNK, chunk, 0)


def _build_gather():
    return pl.kernel(
        _sc_gather_body,
        out_type=[jax.ShapeDtypeStruct((EP, H), jnp.float32),
                  jax.ShapeDtypeStruct((EP, H), jnp.float32)],
        mesh=plsc.VectorSubcoreMesh(core_axis_name="c", subcore_axis_name="s"),
        scratch_types=[pltpu.VMEM((NCHUNK, CH), jnp.int32),
                       pltpu.VMEM((NCHUNK, CH), jnp.int32),
                       pltpu.VMEM((CH, H), jnp.float32),
                       pltpu.VMEM((CH, H), jnp.float32),
                       pltpu.SemaphoreType.DMA,
                       pltpu.SemaphoreType.DMA],
    )


def _sc_scatter_body(m2_hbm, dsti, zeros_hbm, part_out, idxd, mbuf, acc):
    c = lax.axis_index("c")
    s = lax.axis_index("s")
    wid = s * NC + c
    pltpu.sync_copy(zeros_hbm, acc.at[pl.ds(s * TPS, TPS), :])
    plsc.subcore_barrier()
    pltpu.sync_copy(dsti.at[wid], idxd)

    def chunk(j, carry):
        pltpu.sync_copy(m2_hbm.at[pl.ds(wid * EPW + j * CH, CH), :], mbuf)
        pltpu.sync_copy(mbuf, acc.at[idxd.at[j]], add=True)
        return carry

    lax.fori_loop(0, NCHUNK, chunk, 0)
    plsc.subcore_barrier()
    pltpu.sync_copy(acc.at[pl.ds(s * TPS, TPS), :],
                    part_out.at[c, pl.ds(s * TPS, TPS), :])


def _build_scatter():
    return pl.kernel(
        _sc_scatter_body,
        out_type=jax.ShapeDtypeStruct((NC, NPT, H), jnp.float32),
        mesh=plsc.VectorSubcoreMesh(core_axis_name="c", subcore_axis_name="s"),
        scratch_types=[pltpu.VMEM((NCHUNK, CH), jnp.int32),
                       pltpu.VMEM((CH, H), jnp.float32),
                       pltpu.VMEM_SHARED((NPT, H), jnp.float32)],
    )


def _sc_count_body(dsti, ones_hbm, zeros_hbm, cnt_out, idxd, onesb, acc):
    c = lax.axis_index("c")
    s = lax.axis_index("s")
    wid = s * NC + c
    pltpu.sync_copy(zeros_hbm, acc.at[pl.ds(s * TPS, TPS), :])
    pltpu.sync_copy(ones_hbm, onesb)
    plsc.subcore_barrier()
    pltpu.sync_copy(dsti.at[wid], idxd)

    def chunk(j, carry):
        pltpu.sync_copy(onesb, acc.at[idxd.at[j]], add=True)
        return carry

    lax.fori_loop(0, NCHUNK, chunk, 0)
    plsc.subcore_barrier()
    pltpu.sync_copy(acc.at[pl.ds(s * TPS, TPS), :],
                    cnt_out.at[c, pl.ds(s * TPS, TPS), :])


def _build_count():
    return pl.kernel(
        _sc_count_body,
        out_type=jax.ShapeDtypeStruct((NC, NPT, H), jnp.float32),
        mesh=plsc.VectorSubcoreMesh(core_axis_name="c", subcore_axis_name="s"),
        scratch_types=[pltpu.VMEM((NCHUNK, CH), jnp.int32),
                       pltpu.VMEM((CH, H), jnp.float32),
                       pltpu.VMEM_SHARED((NPT, H), jnp.float32)],
    )


def _mm_body(pd_ref, qs_ref, w_ref, b_ref, o_ref):
    g = jnp.maximum(pd_ref[...] + qs_ref[...], 0.0)
    m = jnp.dot(g, w_ref[...], preferred_element_type=jnp.float32, precision=lax.Precision.HIGHEST)
    o_ref[...] = jnp.maximum(m + b_ref[0:1, :], 0.0)


_mm = pl.pallas_call(
    _mm_body,
    grid=(EP // MM_BLK,),
    in_specs=[pl.BlockSpec((MM_BLK, H), lambda i: (i, 0)),
              pl.BlockSpec((MM_BLK, H), lambda i: (i, 0)),
              pl.BlockSpec((H, H), lambda i: (0, 0)),
              pl.BlockSpec((8, H), lambda i: (0, 0))],
    out_specs=pl.BlockSpec((MM_BLK, H), lambda i: (i, 0)),
    out_shape=jax.ShapeDtypeStruct((EP, H), jnp.float32),
)


def _bn_in_kernel(x, g_ref, b_ref):
    mu = jnp.mean(x, axis=0)
    var = jnp.mean((x - mu) ** 2, axis=0)
    return g_ref[0:1, :] * (x - mu) / jnp.sqrt(var + 1e-5) + b_ref[0:1, :]


def _pqk_body(h_ref, scal_ref, a_ref, b2_ref, w4_ref, bm1_ref, p_out, q_out):
    h = h_ref[...]
    u, px, py, vv = (scal_ref[:, 0:1], scal_ref[:, 1:2],
                     scal_ref[:, 2:3], scal_ref[:, 3:4])
    svec = (u * w4_ref[0:1, :] + px * w4_ref[1:2, :] + py * w4_ref[2:3, :])
    p = jnp.dot(h, a_ref[...], preferred_element_type=jnp.float32,
                precision=lax.Precision.HIGHEST)
    p_out[...] = p + svec + vv * w4_ref[3:4, :] + bm1_ref[0:1, :]
    q_out[...] = jnp.dot(h, b2_ref[...], preferred_element_type=jnp.float32,
                         precision=lax.Precision.HIGHEST) - svec


BLKP = NPT // 8

_pqk = pl.pallas_call(
    _pqk_body,
    grid=(8,),
    in_specs=[pl.BlockSpec((BLKP, H), lambda i: (i, 0)),
              pl.BlockSpec((BLKP, 4), lambda i: (i, 0)),
              pl.BlockSpec((H, H), lambda i: (0, 0)),
              pl.BlockSpec((H, H), lambda i: (0, 0)),
              pl.BlockSpec((4, H), lambda i: (0, 0)),
              pl.BlockSpec((8, H), lambda i: (0, 0))],
    out_specs=[pl.BlockSpec((BLKP, H), lambda i: (i, 0)),
               pl.BlockSpec((BLKP, H), lambda i: (i, 0))],
    out_shape=[jax.ShapeDtypeStruct((NPT, H), jnp.float32),
               jax.ShapeDtypeStruct((NPT, H), jnp.float32)],
)


def _pre_body(scal_ref, w1_ref, b1_ref, g1_ref, be1_ref, w2_ref, b2_ref,
              g2_ref, be2_ref, h_out):
    scal = scal_ref[0:N, :]
    u, px, py, vv = (scal[:, 0:1], scal[:, 1:2], scal[:, 2:3], scal[:, 3:4])
    h = (u * w1_ref[0:1, :] + px * w1_ref[1:2, :] + py * w1_ref[2:3, :]
         + vv * w1_ref[3:4, :] + b1_ref[0:1, :])
    h = _bn_in_kernel(h, g1_ref, be1_ref)
    h = jnp.maximum(h, 0.0)
    h = jnp.dot(h, w2_ref[...], preferred_element_type=jnp.float32,
                precision=lax.Precision.HIGHEST) + b2_ref[0:1, :]
    h = _bn_in_kernel(h, g2_ref, be2_ref)
    h_out[0:N, :] = h
    h_out[N:NPT, :] = jnp.zeros((NPT - N, H), jnp.float32)


_pre = pl.pallas_call(
    _pre_body,
    out_shape=jax.ShapeDtypeStruct((NPT, H), jnp.float32),
)


def _update_h(h_in_ref, parts_ref, cnts_ref, scal_ref, wu1h_ref, wu1a_ref,
              wu1v_ref, bu1_ref, wu2_ref, bu2_ref, g_ref, be_ref):
    h = h_in_ref[0:N, :]
    agg = parts_ref[0, 0:N, :] + parts_ref[1, 0:N, :]
    cnt = cnts_ref[0:N, 0:1]
    agg = agg / jnp.maximum(cnt, 1.0)
    vv = scal_ref[0:N, 3:4]
    upd = (jnp.dot(h, wu1h_ref[...], preferred_element_type=jnp.float32, precision=lax.Precision.HIGHEST)
           + jnp.dot(agg, wu1a_ref[...], preferred_element_type=jnp.float32, precision=lax.Precision.HIGHEST)
           + vv * wu1v_ref[0:1, :] + bu1_ref[0:1, :])
    upd = jnp.maximum(upd, 0.0)
    upd = jnp.dot(upd, wu2_ref[...], preferred_element_type=jnp.float32, precision=lax.Precision.HIGHEST) + bu2_ref[0:1, :]
    upd = jnp.maximum(upd, 0.0)
    return _bn_in_kernel(h + upd, g_ref, be_ref)


def _node_body(h_in_ref, parts_ref, cnts_ref, scal_ref, wu1h_ref, wu1a_ref,
               wu1v_ref, bu1_ref, wu2_ref, bu2_ref, g_ref, be_ref, h_out):
    hn = _update_h(h_in_ref, parts_ref, cnts_ref, scal_ref, wu1h_ref,
                   wu1a_ref, wu1v_ref, bu1_ref, wu2_ref, bu2_ref, g_ref, be_ref)
    h_out[0:N, :] = hn
    h_out[N:NPT, :] = jnp.zeros((NPT - N, H), jnp.float32)


_node = pl.pallas_call(
    _node_body,
    out_shape=jax.ShapeDtypeStruct((NPT, H), jnp.float32),
)


def _final_body(h_in_ref, parts_ref, cnts_ref, scal_ref, wu1h_ref, wu1a_ref,
                wu1v_ref, bu1_ref, wu2_ref, bu2_ref, g_ref, be_ref,
                w1c_ref, b1c_ref, w2c_ref, b2c_ref, w3c_ref, b3c_ref, out_ref):
    hn = _update_h(h_in_ref, parts_ref, cnts_ref, scal_ref, wu1h_ref,
                   wu1a_ref, wu1v_ref, bu1_ref, wu2_ref, bu2_ref, g_ref, be_ref)
    z1 = jnp.maximum(jnp.dot(hn, w1c_ref[...], preferred_element_type=jnp.float32, precision=lax.Precision.HIGHEST)
                     + b1c_ref[0:1, :], 0.0)
    z2 = jnp.maximum(jnp.dot(z1, w2c_ref[...], preferred_element_type=jnp.float32, precision=lax.Precision.HIGHEST)
                     + b2c_ref[0:1, :], 0.0)
    z3 = jnp.dot(z2, w3c_ref[...], preferred_element_type=jnp.float32, precision=lax.Precision.HIGHEST)
    out_ref[...] = (z3 + b3c_ref[0:1, :]) * 0.01


_final = pl.pallas_call(
    _final_body,
    out_shape=jax.ShapeDtypeStruct((N, H), jnp.float32),
)


def _conv_maps():
    r1, c1, o1, k1 = [], [], [], []
    for o in range(4):
        for j in range(38):
            for k in range(16):
                r1.append(3 * j + k); c1.append(o * 38 + j); o1.append(o); k1.append(k)
    r2, c2, o2, i2, k2 = [], [], [], [], []
    for o in range(8):
        for j in range(9):
            for i in range(4):
                for k in range(12):
                    r2.append(i * 38 + 3 * j + k); c2.append(o * 9 + j)
                    o2.append(o); i2.append(i); k2.append(k)
    r3, i3, k3 = [], [], []
    for i in range(8):
        for k in range(8):
            r3.append(i * 9 + k); i3.append(i); k3.append(k)
    f = lambda v: np.asarray(v, np.int32)
    return ((f(r1), f(c1), f(o1), f(k1)),
            (f(r2), f(c2), f(o2), f(i2), f(k2)),
            (f(r3), f(i3), f(k3)))


_C1, _C2, _C3 = _conv_maps()


def _row8(v):
    return jnp.broadcast_to(v.reshape(1, -1), (8, v.shape[-1]))


def kernel(x, pos, edge_index, batch, params):
    f32 = jnp.float32
    scal = jnp.concatenate(
        [x[:, 0:1], pos[:, 1:2], pos[:, 2:3], pos[:, 0:1]], axis=1).astype(f32)
    scal = jnp.concatenate([scal, jnp.zeros((NPT - N, 4), f32)], axis=0)

    pad = EP - E
    srcp = jnp.concatenate([edge_index[0], jnp.full((pad,), N, jnp.int32)])
    dstp = jnp.concatenate([edge_index[1], jnp.full((pad,), N, jnp.int32)])
    src3 = srcp.reshape(NW, NCHUNK, CH)
    dst3 = dstp.reshape(NW, NCHUNK, CH)

    zeros_h = jnp.zeros((TPS, H), f32)
    ones_c = jnp.ones((CH, H), f32)

    emb = params['emb']
    layers = params['layers']
    conv = params['conv']

    def msg_split(lp):
        wm1 = lp['wm1']
        return (wm1[0:H], wm1[H:2 * H], wm1[2 * H:2 * H + 4], _row8(lp['bm1']))

    h = _pre(scal, emb['w1'], _row8(emb['b1']), _row8(emb['g1']),
             _row8(emb['be1']), emb['w2'], _row8(emb['b2']),
             _row8(emb['g2']), _row8(emb['be2']))

    cparts = _build_count()(dst3, ones_c, zeros_h)
    cnts = cparts[0, :, 0:1] + cparts[1, :, 0:1]

    w1c = jnp.zeros((H, 256), f32).at[_C1[0], _C1[1]].set(
        conv['w1'][_C1[2], 0, _C1[3]])
    b1c = jnp.zeros((256,), f32).at[np.arange(152, dtype=np.int32)].set(
        jnp.repeat(conv['b1'], 38))
    w2c = jnp.zeros((256, H), f32).at[_C2[0], _C2[1]].set(
        conv['w2'][_C2[2], _C2[3], _C2[4]])
    b2c = jnp.zeros((H,), f32).at[np.arange(72, dtype=np.int32)].set(
        jnp.repeat(conv['b2'], 9))
    w3c = jnp.zeros((H, H), f32).at[_C3[0], 0].set(conv['w3'][0, _C3[1], _C3[2]])
    b3c = jnp.broadcast_to(conv['b3'].reshape(1, 1), (8, H))

    _gather = _build_gather()
    _scatter = _build_scatter()

    out_full = None
    for i in range(NLAYERS):
        lp = layers[i]
        ai, bi, w4i, bm1i = msg_split(lp)
        p, q = _pqk(h, scal, ai, bi, w4i, bm1i)
        pd, qs = _gather(p, q, dst3, src3)
        m2 = _mm(pd, qs, lp['wm2'], _row8(lp['bm2']))
        parts = _scatter(m2, dst3, zeros_h)
        wu1 = lp['wu1']
        common = (h, parts, cnts, scal, wu1[0:H], wu1[H:2 * H],
                  wu1[2 * H:2 * H + 1], _row8(lp['bu1']), lp['wu2'],
                  _row8(lp['bu2']), _row8(lp['g']), _row8(lp['be']))
        if i < NLAYERS - 1:
            h = _node(*common)
        else:
            out_full = _final(*common, w1c, _row8(b1c), w2c, _row8(b2c),
                              w3c, b3c)
    return out_full[:, 0:1]


# speedup vs baseline: 6.0880x; 1.1622x over previous
import numpy as np
import jax
import jax.numpy as jnp
from jax import lax
from jax.experimental import pallas as pl
from jax.experimental.pallas import tpu as pltpu
from jax.experimental.pallas import tpu_sc as plsc

N = 10000
E = 320000
H = 128
NC = 2
NS = 16
NW = NC * NS
CH = 128
NCHUNK = 79
EPW = NCHUNK * CH
EP = NW * EPW
NPT = 10112
TPS = NPT // NS
MM_BLK = 2048
NLAYERS = 6


def _sc_gather_body(p_hbm, q_hbm, dsti, srci, pd_out, qs_out,
                    idxd, idxs, ba0, bb0, ba1, bb1,
                    sa0, sb0, sa1, sb1, so0, so1):
    c = lax.axis_index("c")
    s = lax.axis_index("s")
    wid = s * NC + c
    base = wid * EPW
    pltpu.sync_copy(dsti.at[wid], idxd)
    pltpu.sync_copy(srci.at[wid], idxs)

    slots = ((ba0, bb0, sa0, sb0, so0), (ba1, bb1, sa1, sb1, so1))

    def issue(k, j):
        ba, bb, sa, sb, _ = slots[k]
        pltpu.async_copy(p_hbm.at[idxd.at[j]], ba, sa)
        pltpu.async_copy(q_hbm.at[idxs.at[j]], bb, sb)

    def drain(k, j):
        ba, bb, sa, sb, so = slots[k]
        pltpu.make_async_copy(p_hbm.at[idxd.at[j]], ba, sa).wait()
        pltpu.make_async_copy(q_hbm.at[idxs.at[j]], bb, sb).wait()
        pltpu.async_copy(ba, pd_out.at[pl.ds(base + j * CH, CH), :], so)
        pltpu.async_copy(bb, qs_out.at[pl.ds(base + j * CH, CH), :], so)

    def dstore(k, j):
        ba, bb, _, _, so = slots[k]
        pltpu.make_async_copy(ba, pd_out.at[pl.ds(base + j * CH, CH), :], so).wait()
        pltpu.make_async_copy(bb, qs_out.at[pl.ds(base + j * CH, CH), :], so).wait()

    issue(0, 0)

    def pair(t, carry):
        j = 2 * t
        issue(1, j + 1)
        drain(0, j)
        dstore(0, j)
        issue(0, j + 2)
        drain(1, j + 1)
        dstore(1, j + 1)
        return carry

    lax.fori_loop(0, (NCHUNK - 1) // 2, pair, 0)
    drain(0, NCHUNK - 1)
    dstore(0, NCHUNK - 1)


def _build_gather():
    return pl.kernel(
        _sc_gather_body,
        out_type=[jax.ShapeDtypeStruct((EP, H), jnp.float32),
                  jax.ShapeDtypeStruct((EP, H), jnp.float32)],
        mesh=plsc.VectorSubcoreMesh(core_axis_name="c", subcore_axis_name="s"),
        scratch_types=[pltpu.VMEM((NCHUNK, CH), jnp.int32),
                       pltpu.VMEM((NCHUNK, CH), jnp.int32),
                       pltpu.VMEM((CH, H), jnp.float32),
                       pltpu.VMEM((CH, H), jnp.float32),
                       pltpu.VMEM((CH, H), jnp.float32),
                       pltpu.VMEM((CH, H), jnp.float32),
                       pltpu.SemaphoreType.DMA,
                       pltpu.SemaphoreType.DMA,
                       pltpu.SemaphoreType.DMA,
                       pltpu.SemaphoreType.DMA,
                       pltpu.SemaphoreType.DMA,
                       pltpu.SemaphoreType.DMA],
    )


def _sc_scatter_body(m2_hbm, dsti, zeros_hbm, part_out,
                     idxd, mb0, mb1, acc, sm0, sm1):
    c = lax.axis_index("c")
    s = lax.axis_index("s")
    wid = s * NC + c
    base = wid * EPW
    pltpu.sync_copy(zeros_hbm, acc.at[pl.ds(s * TPS, TPS), :])
    plsc.subcore_barrier()
    pltpu.sync_copy(dsti.at[wid], idxd)

    slots = ((mb0, sm0), (mb1, sm1))

    def issue(k, j):
        mb, sm = slots[k]
        pltpu.async_copy(m2_hbm.at[pl.ds(base + j * CH, CH), :], mb, sm)

    def drain(k, j):
        mb, sm = slots[k]
        pltpu.make_async_copy(m2_hbm.at[pl.ds(base + j * CH, CH), :], mb, sm).wait()
        pltpu.sync_copy(mb, acc.at[idxd.at[j]], add=True)

    issue(0, 0)

    def pair(t, carry):
        j = 2 * t
        issue(1, j + 1)
        drain(0, j)
        issue(0, j + 2)
        drain(1, j + 1)
        return carry

    lax.fori_loop(0, (NCHUNK - 1) // 2, pair, 0)
    drain(0, NCHUNK - 1)
    plsc.subcore_barrier()
    pltpu.sync_copy(acc.at[pl.ds(s * TPS, TPS), :],
                    part_out.at[c, pl.ds(s * TPS, TPS), :])


def _build_scatter():
    return pl.kernel(
        _sc_scatter_body,
        out_type=jax.ShapeDtypeStruct((NC, NPT, H), jnp.float32),
        mesh=plsc.VectorSubcoreMesh(core_axis_name="c", subcore_axis_name="s"),
        scratch_types=[pltpu.VMEM((NCHUNK, CH), jnp.int32),
                       pltpu.VMEM((CH, H), jnp.float32),
                       pltpu.VMEM((CH, H), jnp.float32),
                       pltpu.VMEM_SHARED((NPT, H), jnp.float32),
                       pltpu.SemaphoreType.DMA,
                       pltpu.SemaphoreType.DMA],
    )


def _sc_count_body(dsti, ones_hbm, zeros_hbm, cnt_out, idxd, onesb, acc):
    c = lax.axis_index("c")
    s = lax.axis_index("s")
    wid = s * NC + c
    pltpu.sync_copy(zeros_hbm, acc.at[pl.ds(s * TPS, TPS), :])
    pltpu.sync_copy(ones_hbm, onesb)
    plsc.subcore_barrier()
    pltpu.sync_copy(dsti.at[wid], idxd)

    def chunk(j, carry):
        pltpu.sync_copy(onesb, acc.at[idxd.at[j]], add=True)
        return carry

    lax.fori_loop(0, NCHUNK, chunk, 0)
    plsc.subcore_barrier()
    pltpu.sync_copy(acc.at[pl.ds(s * TPS, TPS), :],
                    cnt_out.at[c, pl.ds(s * TPS, TPS), :])


def _build_count():
    return pl.kernel(
        _sc_count_body,
        out_type=jax.ShapeDtypeStruct((NC, NPT, H), jnp.float32),
        mesh=plsc.VectorSubcoreMesh(core_axis_name="c", subcore_axis_name="s"),
        scratch_types=[pltpu.VMEM((NCHUNK, CH), jnp.int32),
                       pltpu.VMEM((CH, H), jnp.float32),
                       pltpu.VMEM_SHARED((NPT, H), jnp.float32)],
    )


def _mm_body(pd_ref, qs_ref, w_ref, b_ref, o_ref):
    g = jnp.maximum(pd_ref[...] + qs_ref[...], 0.0)
    m = jnp.dot(g, w_ref[...], preferred_element_type=jnp.float32, precision=lax.Precision.HIGHEST)
    o_ref[...] = jnp.maximum(m + b_ref[0:1, :], 0.0)


_mm = pl.pallas_call(
    _mm_body,
    grid=(EP // MM_BLK,),
    in_specs=[pl.BlockSpec((MM_BLK, H), lambda i: (i, 0)),
              pl.BlockSpec((MM_BLK, H), lambda i: (i, 0)),
              pl.BlockSpec((H, H), lambda i: (0, 0)),
              pl.BlockSpec((8, H), lambda i: (0, 0))],
    out_specs=pl.BlockSpec((MM_BLK, H), lambda i: (i, 0)),
    out_shape=jax.ShapeDtypeStruct((EP, H), jnp.float32),
)


def _bn_in_kernel(x, g_ref, b_ref):
    mu = jnp.mean(x, axis=0)
    var = jnp.mean((x - mu) ** 2, axis=0)
    return g_ref[0:1, :] * (x - mu) / jnp.sqrt(var + 1e-5) + b_ref[0:1, :]


def _pqk_body(h_ref, scal_ref, a_ref, b2_ref, w4_ref, bm1_ref, p_out, q_out):
    h = h_ref[...]
    u, px, py, vv = (scal_ref[:, 0:1], scal_ref[:, 1:2],
                     scal_ref[:, 2:3], scal_ref[:, 3:4])
    svec = (u * w4_ref[0:1, :] + px * w4_ref[1:2, :] + py * w4_ref[2:3, :])
    p = jnp.dot(h, a_ref[...], preferred_element_type=jnp.float32,
                precision=lax.Precision.HIGHEST)
    p_out[...] = p + svec + vv * w4_ref[3:4, :] + bm1_ref[0:1, :]
    q_out[...] = jnp.dot(h, b2_ref[...], preferred_element_type=jnp.float32,
                         precision=lax.Precision.HIGHEST) - svec


BLKP = NPT // 8

_pqk = pl.pallas_call(
    _pqk_body,
    grid=(8,),
    in_specs=[pl.BlockSpec((BLKP, H), lambda i: (i, 0)),
              pl.BlockSpec((BLKP, 4), lambda i: (i, 0)),
              pl.BlockSpec((H, H), lambda i: (0, 0)),
              pl.BlockSpec((H, H), lambda i: (0, 0)),
              pl.BlockSpec((4, H), lambda i: (0, 0)),
              pl.BlockSpec((8, H), lambda i: (0, 0))],
    out_specs=[pl.BlockSpec((BLKP, H), lambda i: (i, 0)),
               pl.BlockSpec((BLKP, H), lambda i: (i, 0))],
    out_shape=[jax.ShapeDtypeStruct((NPT, H), jnp.float32),
               jax.ShapeDtypeStruct((NPT, H), jnp.float32)],
)


def _pre_body(scal_ref, w1_ref, b1_ref, g1_ref, be1_ref, w2_ref, b2_ref,
              g2_ref, be2_ref, h_out):
    scal = scal_ref[0:N, :]
    u, px, py, vv = (scal[:, 0:1], scal[:, 1:2], scal[:, 2:3], scal[:, 3:4])
    h = (u * w1_ref[0:1, :] + px * w1_ref[1:2, :] + py * w1_ref[2:3, :]
         + vv * w1_ref[3:4, :] + b1_ref[0:1, :])
    h = _bn_in_kernel(h, g1_ref, be1_ref)
    h = jnp.maximum(h, 0.0)
    h = jnp.dot(h, w2_ref[...], preferred_element_type=jnp.float32,
                precision=lax.Precision.HIGHEST) + b2_ref[0:1, :]
    h = _bn_in_kernel(h, g2_ref, be2_ref)
    h_out[0:N, :] = h
    h_out[N:NPT, :] = jnp.zeros((NPT - N, H), jnp.float32)


_pre = pl.pallas_call(
    _pre_body,
    out_shape=jax.ShapeDtypeStruct((NPT, H), jnp.float32),
)


def _update_h(h_in_ref, parts_ref, cnts_ref, scal_ref, wu1h_ref, wu1a_ref,
              wu1v_ref, bu1_ref, wu2_ref, bu2_ref, g_ref, be_ref):
    h = h_in_ref[0:N, :]
    agg = parts_ref[0, 0:N, :] + parts_ref[1, 0:N, :]
    cnt = cnts_ref[0:N, 0:1]
    agg = agg / jnp.maximum(cnt, 1.0)
    vv = scal_ref[0:N, 3:4]
    upd = (jnp.dot(h, wu1h_ref[...], preferred_element_type=jnp.float32, precision=lax.Precision.HIGHEST)
           + jnp.dot(agg, wu1a_ref[...], preferred_element_type=jnp.float32, precision=lax.Precision.HIGHEST)
           + vv * wu1v_ref[0:1, :] + bu1_ref[0:1, :])
    upd = jnp.maximum(upd, 0.0)
    upd = jnp.dot(upd, wu2_ref[...], preferred_element_type=jnp.float32, precision=lax.Precision.HIGHEST) + bu2_ref[0:1, :]
    upd = jnp.maximum(upd, 0.0)
    return _bn_in_kernel(h + upd, g_ref, be_ref)


def _node_body(h_in_ref, parts_ref, cnts_ref, scal_ref, wu1h_ref, wu1a_ref,
               wu1v_ref, bu1_ref, wu2_ref, bu2_ref, g_ref, be_ref, h_out):
    hn = _update_h(h_in_ref, parts_ref, cnts_ref, scal_ref, wu1h_ref,
                   wu1a_ref, wu1v_ref, bu1_ref, wu2_ref, bu2_ref, g_ref, be_ref)
    h_out[0:N, :] = hn
    h_out[N:NPT, :] = jnp.zeros((NPT - N, H), jnp.float32)


_node = pl.pallas_call(
    _node_body,
    out_shape=jax.ShapeDtypeStruct((NPT, H), jnp.float32),
)


def _final_body(h_in_ref, parts_ref, cnts_ref, scal_ref, wu1h_ref, wu1a_ref,
                wu1v_ref, bu1_ref, wu2_ref, bu2_ref, g_ref, be_ref,
                w1c_ref, b1c_ref, w2c_ref, b2c_ref, w3c_ref, b3c_ref, out_ref):
    hn = _update_h(h_in_ref, parts_ref, cnts_ref, scal_ref, wu1h_ref,
                   wu1a_ref, wu1v_ref, bu1_ref, wu2_ref, bu2_ref, g_ref, be_ref)
    z1 = jnp.maximum(jnp.dot(hn, w1c_ref[...], preferred_element_type=jnp.float32, precision=lax.Precision.HIGHEST)
                     + b1c_ref[0:1, :], 0.0)
    z2 = jnp.maximum(jnp.dot(z1, w2c_ref[...], preferred_element_type=jnp.float32, precision=lax.Precision.HIGHEST)
                     + b2c_ref[0:1, :], 0.0)
    z3 = jnp.dot(z2, w3c_ref[...], preferred_element_type=jnp.float32, precision=lax.Precision.HIGHEST)
    out_ref[...] = (z3 + b3c_ref[0:1, :]) * 0.01


_final = pl.pallas_call(
    _final_body,
    out_shape=jax.ShapeDtypeStruct((N, H), jnp.float32),
)


def _conv_maps():
    r1, c1, o1, k1 = [], [], [], []
    for o in range(4):
        for j in range(38):
            for k in range(16):
                r1.append(3 * j + k); c1.append(o * 38 + j); o1.append(o); k1.append(k)
    r2, c2, o2, i2, k2 = [], [], [], [], []
    for o in range(8):
        for j in range(9):
            for i in range(4):
                for k in range(12):
                    r2.append(i * 38 + 3 * j + k); c2.append(o * 9 + j)
                    o2.append(o); i2.append(i); k2.append(k)
    r3, i3, k3 = [], [], []
    for i in range(8):
        for k in range(8):
            r3.append(i * 9 + k); i3.append(i); k3.append(k)
    f = lambda v: np.asarray(v, np.int32)
    return ((f(r1), f(c1), f(o1), f(k1)),
            (f(r2), f(c2), f(o2), f(i2), f(k2)),
            (f(r3), f(i3), f(k3)))


_C1, _C2, _C3 = _conv_maps()


def _row8(v):
    return jnp.broadcast_to(v.reshape(1, -1), (8, v.shape[-1]))


def kernel(x, pos, edge_index, batch, params):
    f32 = jnp.float32
    scal = jnp.concatenate(
        [x[:, 0:1], pos[:, 1:2], pos[:, 2:3], pos[:, 0:1]], axis=1).astype(f32)
    scal = jnp.concatenate([scal, jnp.zeros((NPT - N, 4), f32)], axis=0)

    pad = EP - E
    srcp = jnp.concatenate([edge_index[0], jnp.full((pad,), N, jnp.int32)])
    dstp = jnp.concatenate([edge_index[1], jnp.full((pad,), N, jnp.int32)])
    src3 = srcp.reshape(NW, NCHUNK, CH)
    dst3 = dstp.reshape(NW, NCHUNK, CH)

    zeros_h = jnp.zeros((TPS, H), f32)
    ones_c = jnp.ones((CH, H), f32)

    emb = params['emb']
    layers = params['layers']
    conv = params['conv']

    def msg_split(lp):
        wm1 = lp['wm1']
        return (wm1[0:H], wm1[H:2 * H], wm1[2 * H:2 * H + 4], _row8(lp['bm1']))

    h = _pre(scal, emb['w1'], _row8(emb['b1']), _row8(emb['g1']),
             _row8(emb['be1']), emb['w2'], _row8(emb['b2']),
             _row8(emb['g2']), _row8(emb['be2']))

    cparts = _build_count()(dst3, ones_c, zeros_h)
    cnts = cparts[0, :, 0:1] + cparts[1, :, 0:1]

    w1c = jnp.zeros((H, 256), f32).at[_C1[0], _C1[1]].set(
        conv['w1'][_C1[2], 0, _C1[3]])
    b1c = jnp.zeros((256,), f32).at[np.arange(152, dtype=np.int32)].set(
        jnp.repeat(conv['b1'], 38))
    w2c = jnp.zeros((256, H), f32).at[_C2[0], _C2[1]].set(
        conv['w2'][_C2[2], _C2[3], _C2[4]])
    b2c = jnp.zeros((H,), f32).at[np.arange(72, dtype=np.int32)].set(
        jnp.repeat(conv['b2'], 9))
    w3c = jnp.zeros((H, H), f32).at[_C3[0], 0].set(conv['w3'][0, _C3[1], _C3[2]])
    b3c = jnp.broadcast_to(conv['b3'].reshape(1, 1), (8, H))

    _gather = _build_gather()
    _scatter = _build_scatter()

    out_full = None
    for i in range(NLAYERS):
        lp = layers[i]
        ai, bi, w4i, bm1i = msg_split(lp)
        p, q = _pqk(h, scal, ai, bi, w4i, bm1i)
        pd, qs = _gather(p, q, dst3, src3)
        m2 = _mm(pd, qs, lp['wm2'], _row8(lp['bm2']))
        parts = _scatter(m2, dst3, zeros_h)
        wu1 = lp['wu1']
        common = (h, parts, cnts, scal, wu1[0:H], wu1[H:2 * H],
                  wu1[2 * H:2 * H + 1], _row8(lp['bu1']), lp['wu2'],
                  _row8(lp['bu2']), _row8(lp['g']), _row8(lp['be']))
        if i < NLAYERS - 1:
            h = _node(*common)
        else:
            out_full = _final(*common, w1c, _row8(b1c), w2c, _row8(b2c),
                              w3c, b3c)
    return out_full[:, 0:1]
